# Initial kernel scaffold; baseline (speedup 1.0000x reference)
import jax
import jax.numpy as jnp
from jax.experimental import pallas as pl


def kernel(x, energy_bins, emb):
    raise NotImplementedError("write your pallas kernel here")


# speedup vs baseline: 6.2769x; 6.2769x over previous
import functools

import jax
import jax.numpy as jnp
from jax import lax
from jax.experimental import pallas as pl
from jax.experimental.pallas import tpu as pltpu
from jax.experimental.pallas import tpu_sc as plsc

N_BINS = 256
OUT_DIM = 256
L = 16
NW = 32
B = 16 * 4096
B_W = B // NW
CHUNK = 128
NCH = B_W // CHUNK
PAD_LO = 8
PBINS = 288


def _sc_kernel():
    mesh = plsc.VectorSubcoreMesh(core_axis_name="c", subcore_axis_name="s")

    @functools.partial(
        pl.kernel,
        mesh=mesh,
        out_type=jax.ShapeDtypeStruct((B, OUT_DIM), jnp.float32),
        compiler_params=pltpu.CompilerParams(needs_layout_passes=False),
        scratch_types=[
            pltpu.VMEM((B_W // L, L), jnp.float32),
            pltpu.VMEM((PBINS,), jnp.float32),
            pltpu.VMEM((2, L), jnp.float32),
            pltpu.VMEM((NCH, CHUNK), jnp.int32),
            pltpu.VMEM((CHUNK, OUT_DIM), jnp.float32),
            pltpu.VMEM((CHUNK, OUT_DIM), jnp.float32),
            pltpu.SemaphoreType.DMA,
            pltpu.SemaphoreType.DMA,
        ],
    )
    def k(x_hbm, pbins_hbm, consts_hbm, emb_hbm, out_hbm,
          xv, pbinsv, constsv, idxv, rows0, rows1, sem0, sem1):
        wid = lax.axis_index("s") * 2 + lax.axis_index("c")
        base = wid * B_W

        pltpu.sync_copy(x_hbm.at[wid], xv)
        pltpu.sync_copy(pbins_hbm, pbinsv)
        pltpu.sync_copy(consts_hbm, constsv)

        phi0 = constsv[0]
        inv_s = constsv[1]

        def bucketize_row(c, _):
            for j in range(CHUNK // L):
                xvec = xv[c * (CHUNK // L) + j]
                xi = lax.bitcast_convert_type(xvec, jnp.int32)
                gf = (xi.astype(jnp.float32) - phi0) * inv_s + 0.5
                g = gf.astype(jnp.int32)
                g = jnp.minimum(jnp.maximum(g, 0), N_BINS - 1)
                cnt = g - 3
                for kk in range(6):
                    bv = plsc.load_gather(pbinsv, [g + (PAD_LO - 3 + kk)])
                    cnt = cnt + jnp.where(bv < xvec, 1, 0)
                idxv[c, pl.ds(j * L, L)] = cnt
            return _

        lax.fori_loop(0, NCH, bucketize_row, None)

        bufs = (rows0, rows1)
        sems = (sem0, sem1)
        handles = [None, None]
        handles[0] = pltpu.async_copy(emb_hbm.at[idxv.at[0]], bufs[0], sems[0])
        for c in range(NCH):
            handles[c % 2].wait()
            nxt = c + 1
            if nxt < NCH:
                handles[nxt % 2] = pltpu.async_copy(
                    emb_hbm.at[idxv.at[nxt]], bufs[nxt % 2], sems[nxt % 2])
            pltpu.sync_copy(bufs[c % 2],
                            out_hbm.at[pl.ds(base + c * CHUNK, CHUNK)])

    return k


def kernel(x, energy_bins, emb):
    x3 = x.reshape(NW, B_W // L, L)
    pbins = jnp.concatenate([
        jnp.full((PAD_LO,), -1e38, jnp.float32),
        energy_bins,
        jnp.full((PBINS - PAD_LO - (N_BINS - 1),), 1e38, jnp.float32),
    ])
    bi = lax.bitcast_convert_type(energy_bins, jnp.int32)
    phi0 = bi[0].astype(jnp.float32)
    inv_s = jnp.float32(N_BINS - 2) / (bi[N_BINS - 2].astype(jnp.float32) - phi0)
    consts = jnp.stack([jnp.full((L,), phi0), jnp.full((L,), inv_s)])
    out = _sc_kernel()(x3, pbins, consts, emb)
    return out.reshape(x.shape[0], x.shape[1], OUT_DIM)


# speedup vs baseline: 32.6191x; 5.1967x over previous
import functools

import jax
import jax.numpy as jnp
from jax import lax
from jax.experimental import pallas as pl
from jax.experimental.pallas import tpu as pltpu
from jax.experimental.pallas import tpu_sc as plsc

N_BINS = 256
OUT_DIM = 256
L = 16
NW = 32
B = 16 * 4096
B_W = B // NW
CHUNK = 128
NCH = B_W // CHUNK
PAD_LO = 8
PBINS = 288


NBUF = 3


def _sc_kernel():
    mesh = plsc.VectorSubcoreMesh(core_axis_name="c", subcore_axis_name="s")

    @functools.partial(
        pl.kernel,
        mesh=mesh,
        out_type=jax.ShapeDtypeStruct((B, OUT_DIM), jnp.float32),
        compiler_params=pltpu.CompilerParams(needs_layout_passes=False),
        scratch_types=[
            pltpu.VMEM((B_W // L, L), jnp.float32),
            pltpu.VMEM((PBINS,), jnp.float32),
            pltpu.VMEM((2, L), jnp.float32),
            pltpu.VMEM((NCH, CHUNK), jnp.int32),
            [pltpu.VMEM((CHUNK, OUT_DIM), jnp.float32) for _ in range(NBUF)],
            [pltpu.SemaphoreType.DMA for _ in range(NBUF)],
            [pltpu.SemaphoreType.DMA for _ in range(NBUF)],
        ],
    )
    def k(x_hbm, pbins_hbm, consts_hbm, embt_hbm, out_hbm,
          xv, pbinsv, constsv, idxv, bufs, gsems, wsems):
        wid = lax.axis_index("s") * 2 + lax.axis_index("c")
        base = wid * B_W
        tbase = wid * N_BINS

        pltpu.sync_copy(x_hbm.at[wid], xv)
        pltpu.sync_copy(pbins_hbm, pbinsv)
        pltpu.sync_copy(consts_hbm, constsv)

        phi0 = constsv[0]
        inv_s = constsv[1]

        def bucketize_row(c, _):
            for j in range(CHUNK // L):
                xvec = xv[c * (CHUNK // L) + j]
                xi = lax.bitcast_convert_type(xvec, jnp.int32)
                gf = (xi.astype(jnp.float32) - phi0) * inv_s + 0.5
                g = gf.astype(jnp.int32)
                g = jnp.minimum(jnp.maximum(g, 0), N_BINS - 1)
                cnt = g - 3
                for kk in range(6):
                    bv = plsc.load_gather(pbinsv, [g + (PAD_LO - 3 + kk)])
                    cnt = cnt + jnp.where(bv < xvec, 1, 0)
                idxv[c, pl.ds(j * L, L)] = cnt + tbase
            return _

        lax.fori_loop(0, NCH, bucketize_row, None)

        def gather(c):
            return pltpu.async_copy(
                embt_hbm.at[idxv.at[c]], bufs[c % NBUF], gsems[c % NBUF])

        gh = [None] * NBUF
        wh = [None] * NBUF
        gh[0] = gather(0)
        for c in range(NCH):
            n = c + 1
            if n < NCH:
                bn = n % NBUF
                if wh[bn] is not None:
                    wh[bn].wait()
                gh[bn] = gather(n)
            b = c % NBUF
            gh[b].wait()
            wh[b] = pltpu.async_copy(
                bufs[b], out_hbm.at[pl.ds(base + c * CHUNK, CHUNK)], wsems[b])
        for c in range(NCH - NBUF, NCH):
            wh[c % NBUF].wait()

    return k


def kernel(x, energy_bins, emb):
    x3 = x.reshape(NW, B_W // L, L)
    pbins = jnp.concatenate([
        jnp.full((PAD_LO,), -1e38, jnp.float32),
        energy_bins,
        jnp.full((PBINS - PAD_LO - (N_BINS - 1),), 1e38, jnp.float32),
    ])
    bi = lax.bitcast_convert_type(energy_bins, jnp.int32)
    phi0 = bi[0].astype(jnp.float32)
    inv_s = jnp.float32(N_BINS - 2) / (bi[N_BINS - 2].astype(jnp.float32) - phi0)
    consts = jnp.stack([jnp.full((L,), phi0), jnp.full((L,), inv_s)])
    embt = jnp.tile(emb, (NW, 1))
    out = _sc_kernel()(x3, pbins, consts, embt)
    return out.reshape(x.shape[0], x.shape[1], OUT_DIM)


# speedup vs baseline: 32.8406x; 1.0068x over previous
import functools

import jax
import jax.numpy as jnp
from jax import lax
from jax.experimental import pallas as pl
from jax.experimental.pallas import tpu as pltpu
from jax.experimental.pallas import tpu_sc as plsc

N_BINS = 256
OUT_DIM = 256
L = 16
NW = 32
B = 16 * 4096
B_W = B // NW
CHUNK = 128
NCH = B_W // CHUNK
PAD_LO = 8
PBINS = 288


NBUF = 3


def _sc_kernel():
    mesh = plsc.VectorSubcoreMesh(core_axis_name="c", subcore_axis_name="s")

    @functools.partial(
        pl.kernel,
        mesh=mesh,
        out_type=jax.ShapeDtypeStruct((B, OUT_DIM), jnp.float32),
        compiler_params=pltpu.CompilerParams(needs_layout_passes=False),
        scratch_types=[
            pltpu.VMEM((B_W // L, L), jnp.float32),
            pltpu.VMEM((PBINS,), jnp.float32),
            pltpu.VMEM((2, L), jnp.float32),
            pltpu.VMEM((NCH, CHUNK), jnp.int32),
            [pltpu.VMEM((CHUNK, OUT_DIM), jnp.float32) for _ in range(NBUF)],
            [pltpu.SemaphoreType.DMA for _ in range(NBUF)],
            [pltpu.SemaphoreType.DMA for _ in range(NBUF)],
        ],
    )
    def k(x_hbm, pbins_hbm, consts_hbm, embt_hbm, out_hbm,
          xv, pbinsv, constsv, idxv, bufs, gsems, wsems):
        wid = lax.axis_index("s") * 2 + lax.axis_index("c")
        base = wid * B_W
        tbase = wid * N_BINS

        pltpu.sync_copy(x_hbm.at[wid], xv)
        pltpu.sync_copy(pbins_hbm, pbinsv)
        pltpu.sync_copy(consts_hbm, constsv)

        phi0 = constsv[0]
        inv_s = constsv[1]

        def bucketize_row(c):
            for j in range(CHUNK // L):
                xvec = xv[c * (CHUNK // L) + j]
                xi = lax.bitcast_convert_type(xvec, jnp.int32)
                gf = (xi.astype(jnp.float32) - phi0) * inv_s + 0.5
                g = gf.astype(jnp.int32)
                g = jnp.minimum(jnp.maximum(g, 0), N_BINS - 1)
                cnt = g - 3
                for kk in range(6):
                    bv = plsc.load_gather(pbinsv, [g + (PAD_LO - 3 + kk)])
                    cnt = cnt + jnp.where(bv < xvec, 1, 0)
                idxv[c, pl.ds(j * L, L)] = cnt + tbase

        def gather(c):
            return pltpu.async_copy(
                embt_hbm.at[idxv.at[c]], bufs[c % NBUF], gsems[c % NBUF])

        gh = [None] * NBUF
        wh = [None] * NBUF
        bucketize_row(0)
        gh[0] = gather(0)
        if NCH > 1:
            bucketize_row(1)
        for c in range(NCH):
            n = c + 1
            if n < NCH:
                bn = n % NBUF
                if wh[bn] is not None:
                    wh[bn].wait()
                gh[bn] = gather(n)
            if c + 2 < NCH:
                bucketize_row(c + 2)
            b = c % NBUF
            gh[b].wait()
            wh[b] = pltpu.async_copy(
                bufs[b], out_hbm.at[pl.ds(base + c * CHUNK, CHUNK)], wsems[b])
        for c in range(NCH - NBUF, NCH):
            wh[c % NBUF].wait()

    return k


def kernel(x, energy_bins, emb):
    x3 = x.reshape(NW, B_W // L, L)
    pbins = jnp.concatenate([
        jnp.full((PAD_LO,), -1e38, jnp.float32),
        energy_bins,
        jnp.full((PBINS - PAD_LO - (N_BINS - 1),), 1e38, jnp.float32),
    ])
    bi = lax.bitcast_convert_type(energy_bins, jnp.int32)
    phi0 = bi[0].astype(jnp.float32)
    inv_s = jnp.float32(N_BINS - 2) / (bi[N_BINS - 2].astype(jnp.float32) - phi0)
    consts = jnp.stack([jnp.full((L,), phi0), jnp.full((L,), inv_s)])
    embt = jnp.tile(emb, (NW, 1))
    out = _sc_kernel()(x3, pbins, consts, embt)
    return out.reshape(x.shape[0], x.shape[1], OUT_DIM)
